# baseline (speedup 1.0000x reference)
import functools

import jax
import jax.numpy as jnp
from jax import lax
from jax.experimental import pallas as pl
from jax.experimental.pallas import tpu as pltpu
from jax.experimental.pallas import tpu_sc as plsc

N = 10000
E = 320000
D = 128
C = 40

NP_ = 10240
ROWS_PER_TILE = NP_ // 16
EDGE_K = 128
N_TILES = 32
CHUNKS = 80
EP_ = N_TILES * CHUNKS * EDGE_K
PER_TILE_E = CHUNKS * EDGE_K

_mesh = plsc.VectorSubcoreMesh(core_axis_name="c", subcore_axis_name="s")


@functools.partial(
    pl.kernel,
    out_type=jax.ShapeDtypeStruct((N_TILES * NP_,), jnp.float32),
    mesh=_mesh,
    scratch_types=[
        pltpu.VMEM((CHUNKS, EDGE_K), jnp.int32),
        pltpu.VMEM((NP_,), jnp.float32),
    ],
    compiler_params=pltpu.CompilerParams(needs_layout_passes=False),
)
def _deg_kernel(dst_hbm, zeros_hbm, out_hbm, idx_d, hist):
    c = lax.axis_index("c")
    s = lax.axis_index("s")
    w = c * 16 + s
    ebase = w * PER_TILE_E
    ones16 = jnp.ones((16,), jnp.float32)
    pltpu.sync_copy(zeros_hbm, hist)
    pltpu.sync_copy(dst_hbm.at[pl.ds(w * CHUNKS, CHUNKS)], idx_d)

    def row(j, carry):
        for k in range(EDGE_K // 16):
            v = idx_d[j, pl.ds(k * 16, 16)]
            plsc.addupdate_scatter(hist, [v], ones16)
        return carry

    lax.fori_loop(0, CHUNKS, row, 0)
    pltpu.sync_copy(hist, out_hbm.at[pl.ds(w * NP_, NP_)])


NB = 2
WCH = 16
NWAVES = CHUNKS // WCH
K0 = 32
K1 = 160 - K0

@functools.partial(
    pl.kernel,
    out_type=jax.ShapeDtypeStruct((2 * NP_, D), jnp.float32),
    mesh=_mesh,
    scratch_types=[
        pltpu.VMEM((2, WCH, EDGE_K), jnp.int32),
        pltpu.VMEM((2, WCH, EDGE_K), jnp.int32),
        pltpu.VMEM((NB, EDGE_K, D), jnp.float32),
        pltpu.VMEM_SHARED((NP_, D), jnp.float32),
        pltpu.SemaphoreType.DMA((NB,)),
        pltpu.SemaphoreType.DMA((NB,)),
        pltpu.SemaphoreType.DMA((4,)),
    ],
    compiler_params=pltpu.CompilerParams(needs_layout_passes=False),
)
def _agg_kernel(src_hbm, dst_hbm, y_hbm, zeros_hbm, out_hbm,
                idx_s2, idx_d2, rows, acc, gsem, ssem, isem):
    c = lax.axis_index("c")
    s = lax.axis_index("s")
    base = s * ROWS_PER_TILE
    cbase = jnp.where(c == 0, s * K0, 16 * K0 + s * K1)
    nwaves = jnp.where(c == 0, K0 // WCH, K1 // WCH)

    def stage_idx(wave, slot, sem0):
        pltpu.async_copy(src_hbm.at[pl.ds(cbase + wave * WCH, WCH)],
                         idx_s2.at[slot], isem.at[sem0])
        pltpu.async_copy(dst_hbm.at[pl.ds(cbase + wave * WCH, WCH)],
                         idx_d2.at[slot], isem.at[sem0 + 1])

    def wait_idx(wave, slot, sem0):
        pltpu.make_async_copy(src_hbm.at[pl.ds(cbase + wave * WCH, WCH)],
                              idx_s2.at[slot], isem.at[sem0]).wait()
        pltpu.make_async_copy(dst_hbm.at[pl.ds(cbase + wave * WCH, WCH)],
                              idx_d2.at[slot], isem.at[sem0 + 1]).wait()

    def start_gather(slot, t, b):
        pltpu.async_copy(y_hbm.at[idx_s2.at[slot, t]], rows.at[b], gsem.at[b])

    def wait_gather(slot, t, b):
        pltpu.make_async_copy(y_hbm.at[idx_s2.at[slot, t]], rows.at[b],
                              gsem.at[b]).wait()

    def start_scatter(slot, t, b):
        pltpu.async_copy(rows.at[b], acc.at[idx_d2.at[slot, t]],
                         ssem.at[b], add=True)

    def wait_scatter(slot, t, b):
        pltpu.make_async_copy(rows.at[b], acc.at[idx_d2.at[slot, t]],
                              ssem.at[b]).wait()

    stage_idx(0, 0, 0)
    pltpu.sync_copy(zeros_hbm, rows.at[0])
    for t in range(ROWS_PER_TILE // EDGE_K):
        pltpu.sync_copy(rows.at[0], acc.at[pl.ds(base + t * EDGE_K, EDGE_K)])
    wait_idx(0, 0, 0)
    plsc.subcore_barrier()
    start_gather(0, 0, 0)

    def wave_body(W, carry):
        slot = W % 2
        oslot = (W + 1) % 2

        @pl.when(W + 1 < nwaves)
        def _():
            stage_idx(W + 1, oslot, 2 * oslot)

        for t in range(WCH):
            b = t % NB
            bn = (t + 1) % NB
            if t + 1 < WCH:
                if t == 0:
                    @pl.when(W > 0)
                    def _():
                        wait_scatter(slot, t, bn)
                else:
                    wait_scatter(slot, t, bn)
                start_gather(slot, t + 1, bn)
            else:
                @pl.when(W + 1 < nwaves)
                def _():
                    wait_idx(W + 1, oslot, 2 * oslot)
                    wait_scatter(slot, t, bn)
                    start_gather(oslot, 0, bn)
            wait_gather(slot, t, b)
            start_scatter(slot, t, b)
        return carry

    lax.fori_loop(0, nwaves, wave_body, 0)
    last_slot = (nwaves - 1) % 2
    wait_scatter(last_slot, WCH - 2, (WCH - 2) % NB)
    wait_scatter(last_slot, WCH - 1, (WCH - 1) % NB)
    plsc.subcore_barrier()
    nb_out = ROWS_PER_TILE // EDGE_K
    for t in range(nb_out):
        b = t % NB
        if t >= NB:
            pltpu.make_async_copy(
                rows.at[b],
                out_hbm.at[pl.ds(c * NP_ + base + (t - NB) * EDGE_K, EDGE_K)],
                ssem.at[b]).wait()
        pltpu.sync_copy(acc.at[pl.ds(base + t * EDGE_K, EDGE_K)], rows.at[b])
        pltpu.async_copy(
            rows.at[b],
            out_hbm.at[pl.ds(c * NP_ + base + t * EDGE_K, EDGE_K)],
            ssem.at[b])
    for t in range(nb_out - NB, nb_out):
        b = t % NB
        pltpu.make_async_copy(
            rows.at[b],
            out_hbm.at[pl.ds(c * NP_ + base + t * EDGE_K, EDGE_K)],
            ssem.at[b]).wait()


_BLK = 1280
_GRID = NP_ // _BLK


def _tc1_body(degp_ref, x_ref, w1t_ref, y_ref, dinv_ref):
    deg = jnp.sum(degp_ref[...], axis=1, keepdims=True) + 1.0
    dinv = lax.rsqrt(deg)
    dinv_ref[...] = dinv
    y_ref[...] = lax.dot_general(
        x_ref[...] * dinv, w1t_ref[...],
        (((1,), (0,)), ((), ())), preferred_element_type=jnp.float32)


def _tc2_body(s0_ref, s1_ref, y_ref, dinv_ref, b_ref, wt_ref, o_ref):
    dinv = dinv_ref[...]
    h = (s0_ref[...] + s1_ref[...] + y_ref[...]) * dinv + b_ref[...]
    o_ref[...] = lax.dot_general(
        h * dinv, wt_ref[...],
        (((1,), (0,)), ((), ())), preferred_element_type=jnp.float32)


def _tc3_body(s0_ref, s1_ref, y_ref, dinv_ref, b_ref, wct_ref, bc_ref,
              h_ref, logits_ref):
    dinv = dinv_ref[...]
    h = (s0_ref[...] + s1_ref[...] + y_ref[...]) * dinv + b_ref[...]
    h_ref[...] = h
    logits_ref[...] = lax.dot_general(
        h, wct_ref[...],
        (((1,), (0,)), ((), ())), preferred_element_type=jnp.float32) + bc_ref[...]


def _row_spec(width):
    return pl.BlockSpec((_BLK, width), lambda i: (i, 0))


def _part_spec(width, part):
    off = part * _GRID
    return pl.BlockSpec((_BLK, width), lambda i, _o=off: (_o + i, 0))


def _full_spec(r, w):
    return pl.BlockSpec((r, w), lambda i: (0, 0))


def kernel(edge_index, n_id, feature_vec, W1, b1, W2, b2, Wc, bc):
    del n_id
    src = edge_index[0].astype(jnp.int32)
    dst = edge_index[1].astype(jnp.int32)
    src1 = jnp.pad(src, (0, EP_ - E)).reshape(N_TILES * CHUNKS, EDGE_K)
    pad_dst = N + (jnp.arange(EP_ - E, dtype=jnp.int32) % (NP_ - N))
    dst1 = jnp.concatenate([dst, pad_dst]).reshape(N_TILES * CHUNKS, EDGE_K)

    x = jnp.pad(feature_vec.astype(jnp.float32), ((0, NP_ - N), (0, 0)))
    w1t = W1.T.astype(jnp.float32)
    w2t = W2.T.astype(jnp.float32)
    wct = Wc.T.astype(jnp.float32)
    b1r = b1.reshape(1, D).astype(jnp.float32)
    b2r = b2.reshape(1, D).astype(jnp.float32)
    bcr = bc.reshape(1, C).astype(jnp.float32)

    zeros_rows = jnp.zeros((EDGE_K, D), jnp.float32)

    zeros_hist = jnp.zeros((NP_,), jnp.float32)
    degp = _deg_kernel(dst1, zeros_hist)
    degp_t = degp.reshape(N_TILES, NP_).T

    y1, dinv = pl.pallas_call(
        _tc1_body,
        grid=(_GRID,),
        in_specs=[_row_spec(N_TILES), _row_spec(D), _full_spec(D, D)],
        out_specs=[_row_spec(D), _row_spec(1)],
        out_shape=[jax.ShapeDtypeStruct((NP_, D), jnp.float32),
                   jax.ShapeDtypeStruct((NP_, 1), jnp.float32)],
    )(degp_t, x, w1t)

    s1 = _agg_kernel(src1, dst1, y1, zeros_rows)

    y2 = pl.pallas_call(
        _tc2_body,
        grid=(_GRID,),
        in_specs=[_part_spec(D, 0), _part_spec(D, 1), _row_spec(D),
                  _row_spec(1), _full_spec(1, D), _full_spec(D, D)],
        out_specs=_row_spec(D),
        out_shape=jax.ShapeDtypeStruct((NP_, D), jnp.float32),
    )(s1, s1, y1, dinv, b1r, w2t)

    s2 = _agg_kernel(src1, dst1, y2, zeros_rows)

    h, logits = pl.pallas_call(
        _tc3_body,
        grid=(_GRID,),
        in_specs=[_part_spec(D, 0), _part_spec(D, 1), _row_spec(D),
                  _row_spec(1), _full_spec(1, D), _full_spec(D, C),
                  _full_spec(1, C)],
        out_specs=[_row_spec(D), _row_spec(C)],
        out_shape=[jax.ShapeDtypeStruct((NP_, D), jnp.float32),
                   jax.ShapeDtypeStruct((NP_, C), jnp.float32)],
    )(s2, s2, y2, dinv, b2r, wct, bcr)

    return (logits[:N], h[:N])


# speedup vs baseline: 1.1117x; 1.1117x over previous
import functools

import jax
import jax.numpy as jnp
from jax import lax
from jax.experimental import pallas as pl
from jax.experimental.pallas import tpu as pltpu
from jax.experimental.pallas import tpu_sc as plsc

N = 10000
E = 320000
D = 128
C = 40

NP_ = 10240
ROWS_PER_TILE = NP_ // 16
EDGE_K = 128
N_TILES = 32
CHUNKS = 80
EP_ = N_TILES * CHUNKS * EDGE_K
PER_TILE_E = CHUNKS * EDGE_K

_mesh = plsc.VectorSubcoreMesh(core_axis_name="c", subcore_axis_name="s")


@functools.partial(
    pl.kernel,
    out_type=jax.ShapeDtypeStruct((N_TILES * NP_,), jnp.float32),
    mesh=_mesh,
    scratch_types=[
        pltpu.VMEM((CHUNKS, EDGE_K), jnp.int32),
        pltpu.VMEM((NP_,), jnp.float32),
    ],
    compiler_params=pltpu.CompilerParams(needs_layout_passes=False),
)
def _deg_kernel(dst_hbm, zeros_hbm, out_hbm, idx_d, hist):
    c = lax.axis_index("c")
    s = lax.axis_index("s")
    w = c * 16 + s
    ebase = w * PER_TILE_E
    ones16 = jnp.ones((16,), jnp.float32)
    pltpu.sync_copy(zeros_hbm, hist)
    pltpu.sync_copy(dst_hbm.at[pl.ds(w * CHUNKS, CHUNKS)], idx_d)

    def row(j, carry):
        for k in range(EDGE_K // 16):
            v = idx_d[j, pl.ds(k * 16, 16)]
            plsc.addupdate_scatter(hist, [v], ones16)
        return carry

    lax.fori_loop(0, CHUNKS, row, 0)
    pltpu.sync_copy(hist, out_hbm.at[pl.ds(w * NP_, NP_)])


NB = 2
WCH = 16
NWAVES = CHUNKS // WCH
K0 = 128
K1 = 160 - K0

@functools.partial(
    pl.kernel,
    out_type=jax.ShapeDtypeStruct((2 * NP_, D), jnp.float32),
    mesh=_mesh,
    scratch_types=[
        pltpu.VMEM((2, WCH, EDGE_K), jnp.int32),
        pltpu.VMEM((2, WCH, EDGE_K), jnp.int32),
        pltpu.VMEM((NB, EDGE_K, D), jnp.float32),
        pltpu.VMEM_SHARED((NP_, D), jnp.float32),
        pltpu.SemaphoreType.DMA((NB,)),
        pltpu.SemaphoreType.DMA((NB,)),
        pltpu.SemaphoreType.DMA((4,)),
    ],
    compiler_params=pltpu.CompilerParams(needs_layout_passes=False),
)
def _agg_kernel(src_hbm, dst_hbm, y_hbm, zeros_hbm, out_hbm,
                idx_s2, idx_d2, rows, acc, gsem, ssem, isem):
    c = lax.axis_index("c")
    s = lax.axis_index("s")
    base = s * ROWS_PER_TILE
    cbase = jnp.where(c == 0, s * K0, 16 * K0 + s * K1)
    nwaves = jnp.where(c == 0, K0 // WCH, K1 // WCH)

    def stage_idx(wave, slot, sem0):
        pltpu.async_copy(src_hbm.at[pl.ds(cbase + wave * WCH, WCH)],
                         idx_s2.at[slot], isem.at[sem0])
        pltpu.async_copy(dst_hbm.at[pl.ds(cbase + wave * WCH, WCH)],
                         idx_d2.at[slot], isem.at[sem0 + 1])

    def wait_idx(wave, slot, sem0):
        pltpu.make_async_copy(src_hbm.at[pl.ds(cbase + wave * WCH, WCH)],
                              idx_s2.at[slot], isem.at[sem0]).wait()
        pltpu.make_async_copy(dst_hbm.at[pl.ds(cbase + wave * WCH, WCH)],
                              idx_d2.at[slot], isem.at[sem0 + 1]).wait()

    def start_gather(slot, t, b):
        pltpu.async_copy(y_hbm.at[idx_s2.at[slot, t]], rows.at[b], gsem.at[b])

    def wait_gather(slot, t, b):
        pltpu.make_async_copy(y_hbm.at[idx_s2.at[slot, t]], rows.at[b],
                              gsem.at[b]).wait()

    def start_scatter(slot, t, b):
        pltpu.async_copy(rows.at[b], acc.at[idx_d2.at[slot, t]],
                         ssem.at[b], add=True)

    def wait_scatter(slot, t, b):
        pltpu.make_async_copy(rows.at[b], acc.at[idx_d2.at[slot, t]],
                              ssem.at[b]).wait()

    stage_idx(0, 0, 0)
    pltpu.sync_copy(zeros_hbm, rows.at[0])
    for t in range(ROWS_PER_TILE // EDGE_K):
        pltpu.sync_copy(rows.at[0], acc.at[pl.ds(base + t * EDGE_K, EDGE_K)])
    wait_idx(0, 0, 0)
    plsc.subcore_barrier()
    start_gather(0, 0, 0)

    def wave_body(W, carry):
        slot = W % 2
        oslot = (W + 1) % 2

        @pl.when(W + 1 < nwaves)
        def _():
            stage_idx(W + 1, oslot, 2 * oslot)

        for t in range(WCH):
            b = t % NB
            bn = (t + 1) % NB
            if t + 1 < WCH:
                if t == 0:
                    @pl.when(W > 0)
                    def _():
                        wait_scatter(slot, t, bn)
                else:
                    wait_scatter(slot, t, bn)
                start_gather(slot, t + 1, bn)
            else:
                @pl.when(W + 1 < nwaves)
                def _():
                    wait_idx(W + 1, oslot, 2 * oslot)
                    wait_scatter(slot, t, bn)
                    start_gather(oslot, 0, bn)
            wait_gather(slot, t, b)
            start_scatter(slot, t, b)
        return carry

    lax.fori_loop(0, nwaves, wave_body, 0)
    last_slot = (nwaves - 1) % 2
    wait_scatter(last_slot, WCH - 2, (WCH - 2) % NB)
    wait_scatter(last_slot, WCH - 1, (WCH - 1) % NB)
    plsc.subcore_barrier()
    nb_out = ROWS_PER_TILE // EDGE_K
    for t in range(nb_out):
        b = t % NB
        if t >= NB:
            pltpu.make_async_copy(
                rows.at[b],
                out_hbm.at[pl.ds(c * NP_ + base + (t - NB) * EDGE_K, EDGE_K)],
                ssem.at[b]).wait()
        pltpu.sync_copy(acc.at[pl.ds(base + t * EDGE_K, EDGE_K)], rows.at[b])
        pltpu.async_copy(
            rows.at[b],
            out_hbm.at[pl.ds(c * NP_ + base + t * EDGE_K, EDGE_K)],
            ssem.at[b])
    for t in range(nb_out - NB, nb_out):
        b = t % NB
        pltpu.make_async_copy(
            rows.at[b],
            out_hbm.at[pl.ds(c * NP_ + base + t * EDGE_K, EDGE_K)],
            ssem.at[b]).wait()


_BLK = 1280
_GRID = NP_ // _BLK


def _tc1_body(degp_ref, x_ref, w1t_ref, y_ref, dinv_ref):
    deg = jnp.sum(degp_ref[...], axis=1, keepdims=True) + 1.0
    dinv = lax.rsqrt(deg)
    dinv_ref[...] = dinv
    y_ref[...] = lax.dot_general(
        x_ref[...] * dinv, w1t_ref[...],
        (((1,), (0,)), ((), ())), preferred_element_type=jnp.float32)


def _tc2_body(s0_ref, s1_ref, y_ref, dinv_ref, b_ref, wt_ref, o_ref):
    dinv = dinv_ref[...]
    h = (s0_ref[...] + s1_ref[...] + y_ref[...]) * dinv + b_ref[...]
    o_ref[...] = lax.dot_general(
        h * dinv, wt_ref[...],
        (((1,), (0,)), ((), ())), preferred_element_type=jnp.float32)


def _tc3_body(s0_ref, s1_ref, y_ref, dinv_ref, b_ref, wct_ref, bc_ref,
              h_ref, logits_ref):
    dinv = dinv_ref[...]
    h = (s0_ref[...] + s1_ref[...] + y_ref[...]) * dinv + b_ref[...]
    h_ref[...] = h
    logits_ref[...] = lax.dot_general(
        h, wct_ref[...],
        (((1,), (0,)), ((), ())), preferred_element_type=jnp.float32) + bc_ref[...]


def _row_spec(width):
    return pl.BlockSpec((_BLK, width), lambda i: (i, 0))


def _part_spec(width, part):
    off = part * _GRID
    return pl.BlockSpec((_BLK, width), lambda i, _o=off: (_o + i, 0))


def _full_spec(r, w):
    return pl.BlockSpec((r, w), lambda i: (0, 0))


def kernel(edge_index, n_id, feature_vec, W1, b1, W2, b2, Wc, bc):
    del n_id
    src = edge_index[0].astype(jnp.int32)
    dst = edge_index[1].astype(jnp.int32)
    sc_of_chunk = (jnp.arange(EP_ // EDGE_K, dtype=jnp.int32) >= 16 * K0)
    src1 = (jnp.pad(src, (0, EP_ - E)).reshape(N_TILES * CHUNKS, EDGE_K)
            + sc_of_chunk[:, None] * NP_)
    pad_dst = N + (jnp.arange(EP_ - E, dtype=jnp.int32) % (NP_ - N))
    dst1 = jnp.concatenate([dst, pad_dst]).reshape(N_TILES * CHUNKS, EDGE_K)

    x = jnp.pad(feature_vec.astype(jnp.float32), ((0, NP_ - N), (0, 0)))
    w1t = W1.T.astype(jnp.float32)
    w2t = W2.T.astype(jnp.float32)
    wct = Wc.T.astype(jnp.float32)
    b1r = b1.reshape(1, D).astype(jnp.float32)
    b2r = b2.reshape(1, D).astype(jnp.float32)
    bcr = bc.reshape(1, C).astype(jnp.float32)

    zeros_rows = jnp.zeros((EDGE_K, D), jnp.float32)

    zeros_hist = jnp.zeros((NP_,), jnp.float32)
    degp = _deg_kernel(dst1, zeros_hist)
    degp_t = degp.reshape(N_TILES, NP_).T

    y1, dinv = pl.pallas_call(
        _tc1_body,
        grid=(_GRID,),
        in_specs=[_row_spec(N_TILES), _row_spec(D), _full_spec(D, D)],
        out_specs=[_row_spec(D), _row_spec(1)],
        out_shape=[jax.ShapeDtypeStruct((NP_, D), jnp.float32),
                   jax.ShapeDtypeStruct((NP_, 1), jnp.float32)],
    )(degp_t, x, w1t)

    y1d = jnp.concatenate([y1, y1])
    s1 = _agg_kernel(src1, dst1, y1d, zeros_rows)

    y2 = pl.pallas_call(
        _tc2_body,
        grid=(_GRID,),
        in_specs=[_part_spec(D, 0), _part_spec(D, 1), _row_spec(D),
                  _row_spec(1), _full_spec(1, D), _full_spec(D, D)],
        out_specs=_row_spec(D),
        out_shape=jax.ShapeDtypeStruct((NP_, D), jnp.float32),
    )(s1, s1, y1, dinv, b1r, w2t)

    y2d = jnp.concatenate([y2, y2])
    s2 = _agg_kernel(src1, dst1, y2d, zeros_rows)

    h, logits = pl.pallas_call(
        _tc3_body,
        grid=(_GRID,),
        in_specs=[_part_spec(D, 0), _part_spec(D, 1), _row_spec(D),
                  _row_spec(1), _full_spec(1, D), _full_spec(D, C),
                  _full_spec(1, C)],
        out_specs=[_row_spec(D), _row_spec(C)],
        out_shape=[jax.ShapeDtypeStruct((NP_, D), jnp.float32),
                   jax.ShapeDtypeStruct((NP_, C), jnp.float32)],
    )(s2, s2, y2, dinv, b2r, wct, bcr)

    return (logits[:N], h[:N])


# speedup vs baseline: 1.2093x; 1.0878x over previous
import functools

import jax
import jax.numpy as jnp
from jax import lax
from jax.experimental import pallas as pl
from jax.experimental.pallas import tpu as pltpu
from jax.experimental.pallas import tpu_sc as plsc

N = 10000
E = 320000
D = 128
C = 40

NP_ = 10240
ROWS_PER_TILE = NP_ // 16
EDGE_K = 128
N_TILES = 32
CHUNKS = 80
EP_ = N_TILES * CHUNKS * EDGE_K
PER_TILE_E = CHUNKS * EDGE_K

_mesh = plsc.VectorSubcoreMesh(core_axis_name="c", subcore_axis_name="s")


@functools.partial(
    pl.kernel,
    out_type=jax.ShapeDtypeStruct((N_TILES * NP_,), jnp.float32),
    mesh=_mesh,
    scratch_types=[
        pltpu.VMEM((CHUNKS, EDGE_K), jnp.int32),
        pltpu.VMEM((NP_,), jnp.float32),
    ],
    compiler_params=pltpu.CompilerParams(needs_layout_passes=False),
)
def _deg_kernel(dst_hbm, zeros_hbm, out_hbm, idx_d, hist):
    c = lax.axis_index("c")
    s = lax.axis_index("s")
    w = c * 16 + s
    ebase = w * PER_TILE_E
    ones16 = jnp.ones((16,), jnp.float32)
    pltpu.sync_copy(zeros_hbm, hist)
    pltpu.sync_copy(dst_hbm.at[pl.ds(w * CHUNKS, CHUNKS)], idx_d)

    def row(j, carry):
        for k in range(EDGE_K // 16):
            v = idx_d[j, pl.ds(k * 16, 16)]
            plsc.addupdate_scatter(hist, [v], ones16)
        return carry

    lax.fori_loop(0, CHUNKS, row, 0)
    pltpu.sync_copy(hist, out_hbm.at[pl.ds(w * NP_, NP_)])


NB = 2
WCH = 16
NWAVES = CHUNKS // WCH

@functools.partial(
    pl.kernel,
    out_type=jax.ShapeDtypeStruct((2 * NP_, D), jnp.float32),
    mesh=_mesh,
    scratch_types=[
        pltpu.VMEM((2, WCH, EDGE_K), jnp.int32),
        pltpu.VMEM((2, WCH, EDGE_K), jnp.int32),
        pltpu.VMEM((NB, EDGE_K, D), jnp.float32),
        pltpu.VMEM_SHARED((NP_, D), jnp.float32),
        pltpu.SemaphoreType.DMA((NB,)),
        pltpu.SemaphoreType.DMA((NB,)),
        pltpu.SemaphoreType.DMA((4,)),
    ],
    compiler_params=pltpu.CompilerParams(needs_layout_passes=False),
)
def _agg_kernel(src_hbm, dst_hbm, y_hbm, zeros_hbm, out_hbm,
                idx_s2, idx_d2, rows, acc, gsem, ssem, isem):
    c = lax.axis_index("c")
    s = lax.axis_index("s")
    w = c * 16 + s
    base = s * ROWS_PER_TILE
    cbase = w * CHUNKS

    def stage_idx(wave, slot, sem0):
        pltpu.async_copy(src_hbm.at[pl.ds(cbase + wave * WCH, WCH)],
                         idx_s2.at[slot], isem.at[sem0])
        pltpu.async_copy(dst_hbm.at[pl.ds(cbase + wave * WCH, WCH)],
                         idx_d2.at[slot], isem.at[sem0 + 1])

    def wait_idx(wave, slot, sem0):
        pltpu.make_async_copy(src_hbm.at[pl.ds(cbase + wave * WCH, WCH)],
                              idx_s2.at[slot], isem.at[sem0]).wait()
        pltpu.make_async_copy(dst_hbm.at[pl.ds(cbase + wave * WCH, WCH)],
                              idx_d2.at[slot], isem.at[sem0 + 1]).wait()

    def start_gather(slot, t, b):
        pltpu.async_copy(y_hbm.at[idx_s2.at[slot, t]], rows.at[b], gsem.at[b])

    def wait_gather(slot, t, b):
        pltpu.make_async_copy(y_hbm.at[idx_s2.at[slot, t]], rows.at[b],
                              gsem.at[b]).wait()

    def start_scatter(slot, t, b):
        pltpu.async_copy(rows.at[b], acc.at[idx_d2.at[slot, t]],
                         ssem.at[b], add=True)

    def wait_scatter(slot, t, b):
        pltpu.make_async_copy(rows.at[b], acc.at[idx_d2.at[slot, t]],
                              ssem.at[b]).wait()

    stage_idx(0, 0, 0)
    pltpu.sync_copy(zeros_hbm, rows.at[0])
    for t in range(ROWS_PER_TILE // EDGE_K):
        pltpu.sync_copy(rows.at[0], acc.at[pl.ds(base + t * EDGE_K, EDGE_K)])
    wait_idx(0, 0, 0)
    plsc.subcore_barrier()
    start_gather(0, 0, 0)

    def wave_body(W, carry):
        slot = W % 2
        oslot = (W + 1) % 2

        @pl.when(W + 1 < NWAVES)
        def _():
            stage_idx(W + 1, oslot, 2 * oslot)

        for t in range(WCH):
            b = t % NB
            bn = (t + 1) % NB
            if t + 1 < WCH:
                if t == 0:
                    @pl.when(W > 0)
                    def _():
                        wait_scatter(slot, t, bn)
                else:
                    wait_scatter(slot, t, bn)
                start_gather(slot, t + 1, bn)
            else:
                @pl.when(W + 1 < NWAVES)
                def _():
                    wait_idx(W + 1, oslot, 2 * oslot)
                    wait_scatter(slot, t, bn)
                    start_gather(oslot, 0, bn)
            wait_gather(slot, t, b)
            start_scatter(slot, t, b)
        return carry

    lax.fori_loop(0, NWAVES, wave_body, 0)
    last_slot = (NWAVES - 1) % 2
    wait_scatter(last_slot, WCH - 2, (WCH - 2) % NB)
    wait_scatter(last_slot, WCH - 1, (WCH - 1) % NB)
    plsc.subcore_barrier()
    nb_out = ROWS_PER_TILE // EDGE_K
    for t in range(nb_out):
        b = t % NB
        if t >= NB:
            pltpu.make_async_copy(
                rows.at[b],
                out_hbm.at[pl.ds(c * NP_ + base + (t - NB) * EDGE_K, EDGE_K)],
                ssem.at[b]).wait()
        pltpu.sync_copy(acc.at[pl.ds(base + t * EDGE_K, EDGE_K)], rows.at[b])
        pltpu.async_copy(
            rows.at[b],
            out_hbm.at[pl.ds(c * NP_ + base + t * EDGE_K, EDGE_K)],
            ssem.at[b])
    for t in range(nb_out - NB, nb_out):
        b = t % NB
        pltpu.make_async_copy(
            rows.at[b],
            out_hbm.at[pl.ds(c * NP_ + base + t * EDGE_K, EDGE_K)],
            ssem.at[b]).wait()


_BLK = 1280
_GRID = NP_ // _BLK


def _tc1_body(degp_ref, x_ref, w1t_ref, y_ref, dinv_ref):
    deg = jnp.sum(degp_ref[...], axis=1, keepdims=True) + 1.0
    dinv = lax.rsqrt(deg)
    dinv_ref[...] = dinv
    y_ref[...] = lax.dot_general(
        x_ref[...] * dinv, w1t_ref[...],
        (((1,), (0,)), ((), ())), preferred_element_type=jnp.float32)


def _tc2_body(s0_ref, s1_ref, y_ref, dinv_ref, b_ref, wt_ref, o_ref):
    dinv = dinv_ref[...]
    h = (s0_ref[...] + s1_ref[...] + y_ref[...]) * dinv + b_ref[...]
    o_ref[...] = lax.dot_general(
        h * dinv, wt_ref[...],
        (((1,), (0,)), ((), ())), preferred_element_type=jnp.float32)


def _tc3_body(s0_ref, s1_ref, y_ref, dinv_ref, b_ref, wct_ref, bc_ref,
              h_ref, logits_ref):
    dinv = dinv_ref[...]
    h = (s0_ref[...] + s1_ref[...] + y_ref[...]) * dinv + b_ref[...]
    h_ref[...] = h
    logits_ref[...] = lax.dot_general(
        h, wct_ref[...],
        (((1,), (0,)), ((), ())), preferred_element_type=jnp.float32) + bc_ref[...]


def _row_spec(width):
    return pl.BlockSpec((_BLK, width), lambda i: (i, 0))


def _part_spec(width, part):
    off = part * _GRID
    return pl.BlockSpec((_BLK, width), lambda i, _o=off: (_o + i, 0))


def _full_spec(r, w):
    return pl.BlockSpec((r, w), lambda i: (0, 0))


def kernel(edge_index, n_id, feature_vec, W1, b1, W2, b2, Wc, bc):
    del n_id
    src = edge_index[0].astype(jnp.int32)
    dst = edge_index[1].astype(jnp.int32)
    sc_of_edge = jnp.arange(EP_, dtype=jnp.int32) // (EP_ // 2)
    src1 = (jnp.pad(src, (0, EP_ - E)) + sc_of_edge * NP_
            ).reshape(N_TILES * CHUNKS, EDGE_K)
    pad_dst = N + (jnp.arange(EP_ - E, dtype=jnp.int32) % (NP_ - N))
    dst1 = jnp.concatenate([dst, pad_dst]).reshape(N_TILES * CHUNKS, EDGE_K)

    x = jnp.pad(feature_vec.astype(jnp.float32), ((0, NP_ - N), (0, 0)))
    w1t = W1.T.astype(jnp.float32)
    w2t = W2.T.astype(jnp.float32)
    wct = Wc.T.astype(jnp.float32)
    b1r = b1.reshape(1, D).astype(jnp.float32)
    b2r = b2.reshape(1, D).astype(jnp.float32)
    bcr = bc.reshape(1, C).astype(jnp.float32)

    zeros_rows = jnp.zeros((EDGE_K, D), jnp.float32)

    zeros_hist = jnp.zeros((NP_,), jnp.float32)
    degp = _deg_kernel(dst1, zeros_hist)
    degp_t = degp.reshape(N_TILES, NP_).T

    y1, dinv = pl.pallas_call(
        _tc1_body,
        grid=(_GRID,),
        in_specs=[_row_spec(N_TILES), _row_spec(D), _full_spec(D, D)],
        out_specs=[_row_spec(D), _row_spec(1)],
        out_shape=[jax.ShapeDtypeStruct((NP_, D), jnp.float32),
                   jax.ShapeDtypeStruct((NP_, 1), jnp.float32)],
    )(degp_t, x, w1t)

    y1d = jnp.concatenate([y1, y1])
    s1 = _agg_kernel(src1, dst1, y1d, zeros_rows)

    y2 = pl.pallas_call(
        _tc2_body,
        grid=(_GRID,),
        in_specs=[_part_spec(D, 0), _part_spec(D, 1), _row_spec(D),
                  _row_spec(1), _full_spec(1, D), _full_spec(D, D)],
        out_specs=_row_spec(D),
        out_shape=jax.ShapeDtypeStruct((NP_, D), jnp.float32),
    )(s1, s1, y1, dinv, b1r, w2t)

    y2d = jnp.concatenate([y2, y2])
    s2 = _agg_kernel(src1, dst1, y2d, zeros_rows)

    h, logits = pl.pallas_call(
        _tc3_body,
        grid=(_GRID,),
        in_specs=[_part_spec(D, 0), _part_spec(D, 1), _row_spec(D),
                  _row_spec(1), _full_spec(1, D), _full_spec(D, C),
                  _full_spec(1, C)],
        out_specs=[_row_spec(D), _row_spec(C)],
        out_shape=[jax.ShapeDtypeStruct((NP_, D), jnp.float32),
                   jax.ShapeDtypeStruct((NP_, C), jnp.float32)],
    )(s2, s2, y2, dinv, b2r, wct, bcr)

    return (logits[:N], h[:N])
